# baseline (speedup 1.0000x reference)
import functools

import jax
import jax.numpy as jnp
from jax import lax
from jax.experimental import pallas as pl
from jax.experimental.pallas import tpu as pltpu
from jax.experimental.pallas import tpu_sc as plsc

N = 10000
D = 128
E = 160000
CH = 128
NCORES = 2
NSUB = 16
NW = NCORES * NSUB
CPW = 40
EPAD = NW * CPW * CH
NCHUNK = E // CH
NA = 10008
NAD = 10112
NBUF = 2
RSTRIDE = 624
RWIN = 640


def _sc_scatter_body(feat_hbm, src2_hbm, dst2_hbm, zeros_hbm,
                     out_hbm, srcs_v, dsts_v, rows_v, acc_sh, gsem):
    c = lax.axis_index("c")
    s = lax.axis_index("s")
    wid = c * NSUB + s

    pltpu.sync_copy(src2_hbm.at[pl.ds(wid * CPW, CPW)], srcs_v)
    pltpu.sync_copy(dst2_hbm.at[pl.ds(wid * CPW, CPW)], dsts_v)
    pltpu.sync_copy(zeros_hbm.at[pl.ds(s * RSTRIDE, RWIN)],
                    acc_sh.at[pl.ds(s * RSTRIDE, RWIN)])
    plsc.subcore_barrier()

    count = jnp.clip(NCHUNK - wid * CPW, 0, CPW)

    for b in range(NBUF):
        pltpu.async_copy(feat_hbm.at[srcs_v.at[b]], rows_v.at[b], gsem)

    def step(i, carry):
        j0 = i * NBUF
        for b in range(NBUF):
            j = j0 + b
            pltpu.make_async_copy(feat_hbm.at[pl.ds(0, CH)],
                                  rows_v.at[b], gsem).wait()
            pltpu.sync_copy(rows_v.at[b], acc_sh.at[dsts_v.at[j]], add=True)

            @pl.when(j + NBUF < count)
            def _():
                pltpu.async_copy(feat_hbm.at[srcs_v.at[j + NBUF]],
                                 rows_v.at[b], gsem)

        return carry

    lax.fori_loop(0, count // NBUF, step, 0)
    plsc.subcore_barrier()

    pltpu.sync_copy(acc_sh.at[pl.ds(s * RSTRIDE, RWIN)],
                    out_hbm.at[c, pl.ds(s * RSTRIDE, RWIN)])


@functools.lru_cache(maxsize=1)
def _sc_scatter_kernel():
    return pl.kernel(
        _sc_scatter_body,
        mesh=plsc.VectorSubcoreMesh(core_axis_name="c", subcore_axis_name="s",
                                    num_cores=NCORES, num_subcores=NSUB),
        out_type=jax.ShapeDtypeStruct((NCORES, N, D), jnp.float32),
        scratch_types=[
            pltpu.VMEM((CPW, CH), jnp.int32),
            pltpu.VMEM((CPW, CH), jnp.int32),
            pltpu.VMEM((NBUF, CH, D), jnp.float32),
            pltpu.VMEM_SHARED((NA, D), jnp.float32),
            pltpu.SemaphoreType.DMA,
        ],
        compiler_params=pltpu.CompilerParams(needs_layout_passes=False),
    )


def _sc_scatter(feat, src2, dst2, zeros2d):
    return _sc_scatter_kernel()(feat, src2, dst2, zeros2d)


def _sc_deg_body(dst2_hbm, zrow_hbm, deg_hbm, dsts_v, deg_v):
    c = lax.axis_index("c")
    s = lax.axis_index("s")
    wid = c * NSUB + s

    pltpu.sync_copy(dst2_hbm.at[pl.ds(wid * CPW, CPW)], dsts_v)
    pltpu.sync_copy(zrow_hbm, deg_v)

    zero16 = jnp.zeros((16,), jnp.int32)
    ones16 = jnp.ones((16,), jnp.float32)

    def step(j, carry):
        for k in range(CH // 16):
            idx16 = dsts_v[j, pl.ds(k * 16, 16)]
            plsc.addupdate_scatter(deg_v, [zero16, idx16], ones16)
        return carry

    lax.fori_loop(0, CPW, step, 0)
    pltpu.sync_copy(deg_v, deg_hbm.at[wid])


@functools.lru_cache(maxsize=1)
def _sc_deg_kernel():
    return pl.kernel(
        _sc_deg_body,
        mesh=plsc.VectorSubcoreMesh(core_axis_name="c", subcore_axis_name="s",
                                    num_cores=NCORES, num_subcores=NSUB),
        out_type=jax.ShapeDtypeStruct((NW, 1, NAD), jnp.float32),
        scratch_types=[
            pltpu.VMEM((CPW, CH), jnp.int32),
            pltpu.VMEM((1, NAD), jnp.float32),
        ],
        compiler_params=pltpu.CompilerParams(needs_layout_passes=False),
    )


def _sc_deg(dst2, zrow):
    return _sc_deg_kernel()(dst2, zrow)


BLK = 1024


def _layer_body(p0_ref, p1_ref, degp_ref, xin_ref, wl_ref, wr_ref, b_ref,
                out_ref):
    S = p0_ref[...] + p1_ref[...]
    degp = degp_ref[...].reshape(NW, BLK)
    deg = lax.dot_general(degp, jnp.ones((NW, 1), jnp.float32),
                          (((0,), (0,)), ((), ())),
                          preferred_element_type=jnp.float32)
    agg = S / jnp.maximum(deg, 1.0)
    h = lax.dot_general(agg, wl_ref[...], (((1,), (1,)), ((), ())),
                        preferred_element_type=jnp.float32)
    h = h + lax.dot_general(xin_ref[...], wr_ref[...], (((1,), (1,)), ((), ())),
                            preferred_element_type=jnp.float32)
    out_ref[...] = jnp.maximum(h + b_ref[...], 0.0)


def _layer(p0, p1, degp, xin, wl, wr, b):
    return pl.pallas_call(
        _layer_body,
        grid=(-(-N // BLK),),
        in_specs=[
            pl.BlockSpec((BLK, D), lambda i: (i, 0)),
            pl.BlockSpec((BLK, D), lambda i: (i, 0)),
            pl.BlockSpec((NW, 1, BLK), lambda i: (0, 0, i)),
            pl.BlockSpec((BLK, D), lambda i: (i, 0)),
            pl.BlockSpec((D, D), lambda i: (0, 0)),
            pl.BlockSpec((D, D), lambda i: (0, 0)),
            pl.BlockSpec((1, D), lambda i: (0, 0)),
        ],
        out_specs=pl.BlockSpec((BLK, D), lambda i: (i, 0)),
        out_shape=jax.ShapeDtypeStruct((N, D), jnp.float32),
    )(p0, p1, degp, xin, wl, wr, b)


BR = 200


def _decoder_body(p0_ref, p1_ref, degp_ref, h_ref, wl_ref, wr_ref, b_ref,
                  out_ref, z_scr):
    i = pl.program_id(0)

    @pl.when(i == 0)
    def _():
        S = p0_ref[...] + p1_ref[...]
        degp = degp_ref[...].reshape(NW, NAD)
        deg = lax.dot_general(degp, jnp.ones((NW, 1), jnp.float32),
                              (((0,), (0,)), ((), ())),
                              preferred_element_type=jnp.float32)[:N]
        agg = S / jnp.maximum(deg, 1.0)
        z = lax.dot_general(agg, wl_ref[...], (((1,), (1,)), ((), ())),
                            preferred_element_type=jnp.float32)
        z = z + lax.dot_general(h_ref[...], wr_ref[...],
                                (((1,), (1,)), ((), ())),
                                preferred_element_type=jnp.float32)
        z_scr[...] = jnp.maximum(z + b_ref[...], 0.0)

    zfull = z_scr[...]
    zb = z_scr[pl.ds(i * BR, BR), :]
    logits = lax.dot_general(zb, zfull, (((1,), (1,)), ((), ())),
                             preferred_element_type=jnp.float32)
    m = jnp.max(logits, axis=1, keepdims=True)
    e = jnp.exp(logits - m)
    ssum = jnp.sum(e, axis=1, keepdims=True)
    out_ref[...] = e * (1.0 / ssum)


def _decoder(p0, p1, degp, h, wl, wr, b):
    return pl.pallas_call(
        _decoder_body,
        grid=(N // BR,),
        in_specs=[
            pl.BlockSpec((N, D), lambda i: (0, 0)),
            pl.BlockSpec((N, D), lambda i: (0, 0)),
            pl.BlockSpec((NW, 1, NAD), lambda i: (0, 0, 0)),
            pl.BlockSpec((N, D), lambda i: (0, 0)),
            pl.BlockSpec((D, D), lambda i: (0, 0)),
            pl.BlockSpec((D, D), lambda i: (0, 0)),
            pl.BlockSpec((1, D), lambda i: (0, 0)),
        ],
        out_specs=pl.BlockSpec((BR, N), lambda i: (i, 0)),
        out_shape=jax.ShapeDtypeStruct((N, N), jnp.float32),
        scratch_shapes=[pltpu.VMEM((N, D), jnp.float32)],
    )(p0, p1, degp, h, wl, wr, b)


def kernel(x, edge_index, Wl1, Wr1, b1, Wl2, Wr2, b2):
    src2 = jnp.zeros((EPAD,), jnp.int32).at[:E].set(edge_index[0]).reshape(
        EPAD // CH, CH)
    dst2 = jnp.full((EPAD,), N, jnp.int32).at[:E].set(edge_index[1]).reshape(
        EPAD // CH, CH)
    zeros2d = jnp.zeros((N, D), jnp.float32)
    zrow = jnp.zeros((1, NAD), jnp.float32)
    b1r = b1.reshape(1, D)
    b2r = b2.reshape(1, D)

    degp = _sc_deg(dst2, zrow)
    part1 = _sc_scatter(x, src2, dst2, zeros2d)
    h = _layer(part1[0], part1[1], degp, x, Wl1, Wr1, b1r)
    part2 = _sc_scatter(h, src2, dst2, zeros2d)
    return _decoder(part2[0], part2[1], degp, h, Wl2, Wr2, b2r)


# speedup vs baseline: 1.0048x; 1.0048x over previous
import functools

import jax
import jax.numpy as jnp
from jax import lax
from jax.experimental import pallas as pl
from jax.experimental.pallas import tpu as pltpu
from jax.experimental.pallas import tpu_sc as plsc

N = 10000
D = 128
E = 160000
CH = 128
NCORES = 2
NSUB = 16
NW = NCORES * NSUB
CPW = 40
EPAD = NW * CPW * CH
NCHUNK = E // CH
NA = 10008
NAD = 10112
NBUF = 2
RSTRIDE = 624
RWIN = 640


def _sc_scatter_body(feat_hbm, src2_hbm, dst2_hbm, zeros_hbm,
                     out_hbm, srcs_v, dsts_v, rows_v, acc_sh, gsem):
    c = lax.axis_index("c")
    s = lax.axis_index("s")
    wid = c * NSUB + s

    pltpu.sync_copy(src2_hbm.at[pl.ds(wid * CPW, CPW)], srcs_v)
    pltpu.sync_copy(dst2_hbm.at[pl.ds(wid * CPW, CPW)], dsts_v)
    pltpu.sync_copy(zeros_hbm.at[pl.ds(s * RSTRIDE, RWIN)],
                    acc_sh.at[pl.ds(s * RSTRIDE, RWIN)])
    plsc.subcore_barrier()

    count = jnp.clip(NCHUNK - wid * CPW, 0, CPW)

    for b in range(NBUF):
        pltpu.async_copy(feat_hbm.at[srcs_v.at[b]], rows_v.at[b], gsem)

    def step(i, carry):
        j0 = i * NBUF
        for b in range(NBUF):
            j = j0 + b
            pltpu.make_async_copy(feat_hbm.at[pl.ds(0, CH)],
                                  rows_v.at[b], gsem).wait()
            pltpu.sync_copy(rows_v.at[b], acc_sh.at[dsts_v.at[j]], add=True)

            @pl.when(j + NBUF < count)
            def _():
                pltpu.async_copy(feat_hbm.at[srcs_v.at[j + NBUF]],
                                 rows_v.at[b], gsem)

        return carry

    lax.fori_loop(0, count // NBUF, step, 0)
    plsc.subcore_barrier()

    pltpu.sync_copy(acc_sh.at[pl.ds(s * RSTRIDE, RWIN)],
                    out_hbm.at[c, pl.ds(s * RSTRIDE, RWIN)])


@functools.lru_cache(maxsize=1)
def _sc_scatter_kernel():
    return pl.kernel(
        _sc_scatter_body,
        mesh=plsc.VectorSubcoreMesh(core_axis_name="c", subcore_axis_name="s",
                                    num_cores=NCORES, num_subcores=NSUB),
        out_type=jax.ShapeDtypeStruct((NCORES, N, D), jnp.float32),
        scratch_types=[
            pltpu.VMEM((CPW, CH), jnp.int32),
            pltpu.VMEM((CPW, CH), jnp.int32),
            pltpu.VMEM((NBUF, CH, D), jnp.float32),
            pltpu.VMEM_SHARED((NA, D), jnp.float32),
            pltpu.SemaphoreType.DMA,
        ],
        compiler_params=pltpu.CompilerParams(needs_layout_passes=False),
    )


def _sc_scatter(feat, src2, dst2, zeros2d):
    return _sc_scatter_kernel()(feat, src2, dst2, zeros2d)


def _sc_deg_body(dst2_hbm, zrow_hbm, deg_hbm, dsts_v, deg_v):
    c = lax.axis_index("c")
    s = lax.axis_index("s")
    wid = c * NSUB + s

    pltpu.sync_copy(dst2_hbm.at[pl.ds(wid * CPW, CPW)], dsts_v)
    pltpu.sync_copy(zrow_hbm, deg_v)

    zero16 = jnp.zeros((16,), jnp.int32)
    ones16 = jnp.ones((16,), jnp.float32)

    def step(j, carry):
        for k in range(CH // 16):
            idx16 = dsts_v[j, pl.ds(k * 16, 16)]
            plsc.addupdate_scatter(deg_v, [zero16, idx16], ones16)
        return carry

    lax.fori_loop(0, CPW, step, 0)
    pltpu.sync_copy(deg_v, deg_hbm.at[wid])


@functools.lru_cache(maxsize=1)
def _sc_deg_kernel():
    return pl.kernel(
        _sc_deg_body,
        mesh=plsc.VectorSubcoreMesh(core_axis_name="c", subcore_axis_name="s",
                                    num_cores=NCORES, num_subcores=NSUB),
        out_type=jax.ShapeDtypeStruct((NW, 1, NAD), jnp.float32),
        scratch_types=[
            pltpu.VMEM((CPW, CH), jnp.int32),
            pltpu.VMEM((1, NAD), jnp.float32),
        ],
        compiler_params=pltpu.CompilerParams(needs_layout_passes=False),
    )


def _sc_deg(dst2, zrow):
    return _sc_deg_kernel()(dst2, zrow)


BLK = 1024


def _layer_body(p0_ref, p1_ref, degp_ref, xin_ref, wl_ref, wr_ref, b_ref,
                out_ref):
    S = p0_ref[...] + p1_ref[...]
    degp = degp_ref[...].reshape(NW, BLK)
    deg = lax.dot_general(degp, jnp.ones((NW, 1), jnp.float32),
                          (((0,), (0,)), ((), ())),
                          preferred_element_type=jnp.float32)
    agg = S / jnp.maximum(deg, 1.0)
    h = lax.dot_general(agg, wl_ref[...], (((1,), (1,)), ((), ())),
                        preferred_element_type=jnp.float32)
    h = h + lax.dot_general(xin_ref[...], wr_ref[...], (((1,), (1,)), ((), ())),
                            preferred_element_type=jnp.float32)
    out_ref[...] = jnp.maximum(h + b_ref[...], 0.0)


def _layer(p0, p1, degp, xin, wl, wr, b):
    return pl.pallas_call(
        _layer_body,
        grid=(-(-N // BLK),),
        in_specs=[
            pl.BlockSpec((BLK, D), lambda i: (i, 0)),
            pl.BlockSpec((BLK, D), lambda i: (i, 0)),
            pl.BlockSpec((NW, 1, BLK), lambda i: (0, 0, i)),
            pl.BlockSpec((BLK, D), lambda i: (i, 0)),
            pl.BlockSpec((D, D), lambda i: (0, 0)),
            pl.BlockSpec((D, D), lambda i: (0, 0)),
            pl.BlockSpec((1, D), lambda i: (0, 0)),
        ],
        out_specs=pl.BlockSpec((BLK, D), lambda i: (i, 0)),
        out_shape=jax.ShapeDtypeStruct((N, D), jnp.float32),
    )(p0, p1, degp, xin, wl, wr, b)


BR = 400


def _decoder_body(zfull_ref, zblk_ref, out_ref):
    logits = lax.dot_general(zblk_ref[...], zfull_ref[...],
                             (((1,), (1,)), ((), ())),
                             preferred_element_type=jnp.float32)
    m = jnp.max(logits, axis=1, keepdims=True)
    e = jnp.exp(logits - m)
    ssum = jnp.sum(e, axis=1, keepdims=True)
    out_ref[...] = e * (1.0 / ssum)


def _decoder(z):
    return pl.pallas_call(
        _decoder_body,
        grid=(N // BR,),
        in_specs=[
            pl.BlockSpec((N, D), lambda i: (0, 0)),
            pl.BlockSpec((BR, D), lambda i: (i, 0)),
        ],
        out_specs=pl.BlockSpec((BR, N), lambda i: (i, 0)),
        out_shape=jax.ShapeDtypeStruct((N, N), jnp.float32),
    )(z, z)


def kernel(x, edge_index, Wl1, Wr1, b1, Wl2, Wr2, b2):
    src2 = jnp.zeros((EPAD,), jnp.int32).at[:E].set(edge_index[0]).reshape(
        EPAD // CH, CH)
    dst2 = jnp.full((EPAD,), N, jnp.int32).at[:E].set(edge_index[1]).reshape(
        EPAD // CH, CH)
    zeros2d = jnp.zeros((N, D), jnp.float32)
    zrow = jnp.zeros((1, NAD), jnp.float32)
    b1r = b1.reshape(1, D)
    b2r = b2.reshape(1, D)

    degp = _sc_deg(dst2, zrow)
    part1 = _sc_scatter(x, src2, dst2, zeros2d)
    h = _layer(part1[0], part1[1], degp, x, Wl1, Wr1, b1r)
    part2 = _sc_scatter(h, src2, dst2, zeros2d)
    z = _layer(part2[0], part2[1], degp, h, Wl2, Wr2, b2r)
    return _decoder(z)
